# baseline (speedup 1.0000x reference)
import jax
import jax.numpy as jnp
from jax.experimental import pallas as pl
from jax.experimental.pallas import tpu as pltpu

_ROWS = 3072
_COLS = 8192
_BLK_ROWS = 256
_GRID = _ROWS // _BLK_ROWS
_SCALE = 127.0


def _minmax_body(x_ref, min_ref, max_ref, acc_min, acc_max):
    i = pl.program_id(0)

    @pl.when(i == 0)
    def _init():
        acc_min[...] = jnp.full_like(acc_min, jnp.inf)
        acc_max[...] = jnp.full_like(acc_max, -jnp.inf)

    x = x_ref[...]
    xr = x.reshape(_BLK_ROWS, _COLS // 128, 128)
    acc_min[...] = jnp.minimum(acc_min[...], jnp.min(xr, axis=(0, 1))[None, :])
    acc_max[...] = jnp.maximum(acc_max[...], jnp.max(xr, axis=(0, 1))[None, :])

    @pl.when(i == _GRID - 1)
    def _finish():
        min_ref[0, 0] = jnp.min(acc_min[...])
        max_ref[0, 0] = jnp.max(acc_max[...])


def _quant_body(min_ref, max_ref, x_ref, o_ref):
    x_min = min_ref[0, 0]
    x_max = max_ref[0, 0]
    x = x_ref[...]
    xn = (x - x_min) / (x_max - x_min) * 2.0 - 1.0
    o_ref[...] = jnp.round(xn * _SCALE) / _SCALE


def kernel(x_ll):
    x2 = x_ll.reshape(_ROWS, _COLS)

    x_min, x_max = pl.pallas_call(
        _minmax_body,
        grid=(_GRID,),
        in_specs=[pl.BlockSpec((_BLK_ROWS, _COLS), lambda i: (i, 0))],
        out_specs=[
            pl.BlockSpec(memory_space=pltpu.SMEM),
            pl.BlockSpec(memory_space=pltpu.SMEM),
        ],
        out_shape=[
            jax.ShapeDtypeStruct((1, 1), jnp.float32),
            jax.ShapeDtypeStruct((1, 1), jnp.float32),
        ],
        scratch_shapes=[
            pltpu.VMEM((1, 128), jnp.float32),
            pltpu.VMEM((1, 128), jnp.float32),
        ],
    )(x2)

    q = pl.pallas_call(
        _quant_body,
        grid=(_GRID,),
        in_specs=[
            pl.BlockSpec(memory_space=pltpu.SMEM),
            pl.BlockSpec(memory_space=pltpu.SMEM),
            pl.BlockSpec((_BLK_ROWS, _COLS), lambda i: (i, 0)),
        ],
        out_specs=pl.BlockSpec((_BLK_ROWS, _COLS), lambda i: (i, 0)),
        out_shape=jax.ShapeDtypeStruct((_ROWS, _COLS), jnp.float32),
    )(x_min, x_max, x2)

    return (
        q.reshape(x_ll.shape),
        x_min.reshape(()),
        x_max.reshape(()),
    )


# speedup vs baseline: 3.1209x; 3.1209x over previous
import jax
import jax.numpy as jnp
from jax.experimental import pallas as pl
from jax.experimental.pallas import tpu as pltpu

_B, _C, _H, _W = 4, 96, 256, 256
_BC = 16
_GRID = (_B, _C // _BC)
_NSTEPS = _GRID[0] * _GRID[1]
_SCALE = 127.0


def _minmax_body(x_ref, min_ref, max_ref, acc_min, acc_max):
    step = pl.program_id(0) * pl.num_programs(1) + pl.program_id(1)

    @pl.when(step == 0)
    def _init():
        acc_min[...] = jnp.full_like(acc_min, jnp.inf)
        acc_max[...] = jnp.full_like(acc_max, -jnp.inf)

    x = x_ref[...].reshape(_BC * _H, _W)
    acc_min[...] = jnp.minimum(acc_min[...], jnp.min(x, axis=0, keepdims=True))
    acc_max[...] = jnp.maximum(acc_max[...], jnp.max(x, axis=0, keepdims=True))

    @pl.when(step == _NSTEPS - 1)
    def _finish():
        min_ref[0, 0] = jnp.min(acc_min[...])
        max_ref[0, 0] = jnp.max(acc_max[...])


def _quant_body(min_ref, max_ref, x_ref, o_ref):
    x_min = min_ref[0, 0]
    x_max = max_ref[0, 0]
    x = x_ref[...]
    xn = (x - x_min) / (x_max - x_min) * 2.0 - 1.0
    o_ref[...] = jnp.round(xn * _SCALE) / _SCALE


def kernel(x_ll):
    x_min, x_max = pl.pallas_call(
        _minmax_body,
        grid=_GRID,
        in_specs=[pl.BlockSpec((1, _BC, _H, _W), lambda i, j: (i, j, 0, 0))],
        out_specs=[
            pl.BlockSpec(memory_space=pltpu.SMEM),
            pl.BlockSpec(memory_space=pltpu.SMEM),
        ],
        out_shape=[
            jax.ShapeDtypeStruct((1, 1), jnp.float32),
            jax.ShapeDtypeStruct((1, 1), jnp.float32),
        ],
        scratch_shapes=[
            pltpu.VMEM((1, _W), jnp.float32),
            pltpu.VMEM((1, _W), jnp.float32),
        ],
    )(x_ll)

    q = pl.pallas_call(
        _quant_body,
        grid=_GRID,
        in_specs=[
            pl.BlockSpec(memory_space=pltpu.SMEM),
            pl.BlockSpec(memory_space=pltpu.SMEM),
            pl.BlockSpec((1, _BC, _H, _W), lambda i, j: (i, j, 0, 0)),
        ],
        out_specs=pl.BlockSpec((1, _BC, _H, _W), lambda i, j: (i, j, 0, 0)),
        out_shape=jax.ShapeDtypeStruct((_B, _C, _H, _W), jnp.float32),
    )(x_min, x_max, x_ll)

    return (q, x_min.reshape(()), x_max.reshape(()))
